# Initial kernel scaffold; baseline (speedup 1.0000x reference)
import jax
import jax.numpy as jnp
from jax.experimental import pallas as pl


def kernel(x, device, W1, g1, b1, W2, g2, b2, W3, g3, b3, W4, g4, b4, W5, g5, b5, W6, g6, b6, W7, g7, b7, W8, g8, b8, W9, W10):
    raise NotImplementedError("write your pallas kernel here")


# speedup vs baseline: 4.4211x; 4.4211x over previous
import functools

import jax
import jax.numpy as jnp
from jax import lax
from jax.experimental import pallas as pl
from jax.experimental.pallas import tpu as pltpu
from jax.experimental.pallas import tpu_sc as plsc

N = 2048
K = 20
TN = 256
NT = N // TN
EPS = 1e-5


def _lrelu(v):
    return jnp.where(v >= 0, v, 0.2 * v)


def _gn_affine(sum_c, sumsq_c, gamma, beta, gmat, count):
    gsum = jnp.dot(sum_c, gmat, preferred_element_type=jnp.float32, precision=lax.Precision.HIGHEST)
    gsq = jnp.dot(sumsq_c, gmat, preferred_element_type=jnp.float32, precision=lax.Precision.HIGHEST)
    mean = gsum / count
    var = gsq / count - mean * mean
    inv = lax.rsqrt(var + EPS)
    scale = inv * gamma
    shift = beta - mean * scale
    return scale, shift


def _knn_body(xrow_ref, xall_ref, idx_ref, *, prec):
    b = pl.program_id(0)
    xrow = xrow_ref[0]
    xall = xall_ref[0]
    g = jax.lax.dot_general(xrow, xall, (((1,), (1,)), ((), ())),
                            preferred_element_type=jnp.float32,
                            precision=prec)
    srow = jnp.sum(xrow * xrow, axis=1, keepdims=True)
    sall = jnp.sum(xall * xall, axis=1, keepdims=True)
    work = (2.0 * g - srow) - sall.reshape(1, N)
    iota = lax.broadcasted_iota(jnp.int32, (TN, N), 1)
    cols = []
    for _ in range(K):
        m = jnp.max(work, axis=1, keepdims=True)
        hit = work == m
        cand = jnp.min(jnp.where(hit, iota, N), axis=1)
        cols.append(cand)
        work = jnp.where(iota == cand[:, None], -jnp.inf, work)
    idx = jnp.stack(cols, axis=1)
    idx_ref[0] = idx + b * N


def _knn(xt):
    B, n, d = xt.shape
    prec = None
    return pl.pallas_call(
        functools.partial(_knn_body, prec=prec),
        grid=(B, NT),
        in_specs=[
            pl.BlockSpec((1, TN, d), lambda b, t: (b, t, 0)),
            pl.BlockSpec((1, n, d), lambda b, t: (b, 0, 0)),
        ],
        out_specs=pl.BlockSpec((1, TN, K), lambda b, t: (b, t, 0)),
        out_shape=jax.ShapeDtypeStruct((B, n, K), jnp.int32),
    )(xt, xt)


_GCHUNK = 256


def _sc_gather(table, idx):
    M = idx.shape[0]
    D = table.shape[1]
    info = plsc.get_sparse_core_info()
    nw = info.num_cores * info.num_subcores
    b_per_w = M // nw
    nchunk = b_per_w // _GCHUNK
    mesh = plsc.VectorSubcoreMesh(core_axis_name="c", subcore_axis_name="s")

    @functools.partial(
        pl.kernel,
        mesh=mesh,
        compiler_params=pltpu.CompilerParams(use_tc_tiling_on_sc=False),
        out_type=jax.ShapeDtypeStruct((M, D), jnp.float32),
        scratch_types=[
            pltpu.VMEM((2, _GCHUNK), jnp.int32),
            pltpu.VMEM((2, _GCHUNK, D), jnp.float32),
            pltpu.SemaphoreType.DMA((2,)),
            pltpu.SemaphoreType.DMA((2,)),
        ],
    )
    def gk(table_hbm, idx_hbm, out_hbm, idx_v, rows_v, gsem, osem):
        wid = lax.axis_index("s") * info.num_cores + lax.axis_index("c")
        base = wid * b_per_w

        def out_slice(c):
            return out_hbm.at[pl.ds(base + c * _GCHUNK, _GCHUNK)]

        def fetch(c):
            slot = c % 2
            pltpu.sync_copy(idx_hbm.at[pl.ds(base + c * _GCHUNK, _GCHUNK)],
                            idx_v.at[slot])
            pltpu.async_copy(table_hbm.at[idx_v.at[slot]], rows_v.at[slot],
                             gsem.at[slot])

        fetch(0)
        for c in range(nchunk):
            slot = c % 2
            pltpu.make_async_copy(table_hbm.at[idx_v.at[slot]],
                                  rows_v.at[slot], gsem.at[slot]).wait()
            pltpu.async_copy(rows_v.at[slot], out_slice(c), osem.at[slot])
            if c + 1 < nchunk:
                if c >= 1:
                    pltpu.make_async_copy(rows_v.at[1 - slot],
                                          out_slice(c - 1),
                                          osem.at[1 - slot]).wait()
                fetch(c + 1)
        for cc in (nchunk - 2, nchunk - 1):
            pltpu.make_async_copy(rows_v.at[cc % 2], out_slice(cc),
                                  osem.at[cc % 2]).wait()

    return gk(table, idx)


def _edge_conv1_j(g_ref, c_ref, w1b, j):
    Cin = c_ref.shape[2]
    ctr = c_ref[0]
    nbr = g_ref[0][:, j, :Cin]
    edge = jnp.concatenate([nbr - ctr, ctr], axis=1)
    return jnp.dot(edge.astype(jnp.bfloat16), w1b,
                   preferred_element_type=jnp.float32)


def _edge_stats_body(g_ref, c_ref, w1_ref, stats_ref):
    t = pl.program_id(1)
    C = w1_ref.shape[1]
    w1b = w1_ref[...].astype(jnp.bfloat16)
    s = jnp.zeros((1, C), jnp.float32)
    sq = jnp.zeros((1, C), jnp.float32)
    for j in range(K):
        e = _edge_conv1_j(g_ref, c_ref, w1b, j)
        s = s + jnp.sum(e, axis=0, keepdims=True)
        sq = sq + jnp.sum(e * e, axis=0, keepdims=True)

    @pl.when(t == 0)
    def _():
        stats_ref[0] = jnp.zeros_like(stats_ref[0])

    stats_ref[0, 0:1, :] += s
    stats_ref[0, 1:2, :] += sq


def _edge_stats(G, X, W1):
    B = G.shape[0]
    Gc = G.shape[3]
    Ci = X.shape[2]
    C = W1.shape[1]
    return pl.pallas_call(
        _edge_stats_body,
        grid=(B, NT),
        in_specs=[
            pl.BlockSpec((1, TN, K, Gc), lambda b, t: (b, t, 0, 0)),
            pl.BlockSpec((1, TN, Ci), lambda b, t: (b, t, 0)),
            pl.BlockSpec((2 * Ci, C), lambda b, t: (0, 0)),
        ],
        out_specs=pl.BlockSpec((1, 2, C), lambda b, t: (b, 0, 0)),
        out_shape=jax.ShapeDtypeStruct((B, 2, C), jnp.float32),
    )(G, X, W1)


def _edge_conv2_body(g_ref, c_ref, w1_ref, st_ref, w2_ref, gam_ref, bet_ref,
                     gmat_ref, cmax_ref, cmin_ref, st2_ref):
    t = pl.program_id(1)
    C = w1_ref.shape[1]
    w1b = w1_ref[...].astype(jnp.bfloat16)
    sc1, sh1 = _gn_affine(st_ref[0, 0:1, :], st_ref[0, 1:2, :],
                          gam_ref[...], bet_ref[...], gmat_ref[...],
                          (C // 32) * N * K)
    w2 = w2_ref[...]
    cmax = None
    cmin = None
    ssum = jnp.zeros((1, C), jnp.float32)
    ssq = jnp.zeros((1, C), jnp.float32)
    for j in range(K):
        e = _edge_conv1_j(g_ref, c_ref, w1b, j)
        h = _lrelu(e * sc1 + sh1)
        c2 = jnp.dot(h, w2, preferred_element_type=jnp.float32,
                     precision=lax.Precision.HIGHEST)
        ssum = ssum + jnp.sum(c2, axis=0, keepdims=True)
        ssq = ssq + jnp.sum(c2 * c2, axis=0, keepdims=True)
        cmax = c2 if cmax is None else jnp.maximum(cmax, c2)
        cmin = c2 if cmin is None else jnp.minimum(cmin, c2)
    cmax_ref[0] = cmax
    cmin_ref[0] = cmin

    @pl.when(t == 0)
    def _():
        st2_ref[0] = jnp.zeros_like(st2_ref[0])

    st2_ref[0, 0:1, :] += ssum
    st2_ref[0, 1:2, :] += ssq


def _edge_conv2(G, X, W1, stats1, W2, gamma, beta, gmat):
    B = G.shape[0]
    C = W2.shape[1]
    Ci = X.shape[2]
    Gc = G.shape[3]
    return pl.pallas_call(
        _edge_conv2_body,
        grid=(B, NT),
        in_specs=[
            pl.BlockSpec((1, TN, K, Gc), lambda b, t: (b, t, 0, 0)),
            pl.BlockSpec((1, TN, Ci), lambda b, t: (b, t, 0)),
            pl.BlockSpec((2 * Ci, C), lambda b, t: (0, 0)),
            pl.BlockSpec((1, 2, C), lambda b, t: (b, 0, 0)),
            pl.BlockSpec((C, C), lambda b, t: (0, 0)),
            pl.BlockSpec((1, C), lambda b, t: (0, 0)),
            pl.BlockSpec((1, C), lambda b, t: (0, 0)),
            pl.BlockSpec((C, C), lambda b, t: (0, 0)),
        ],
        out_specs=[
            pl.BlockSpec((1, TN, C), lambda b, t: (b, t, 0)),
            pl.BlockSpec((1, TN, C), lambda b, t: (b, t, 0)),
            pl.BlockSpec((1, 2, C), lambda b, t: (b, 0, 0)),
        ],
        out_shape=[
            jax.ShapeDtypeStruct((B, N, C), jnp.float32),
            jax.ShapeDtypeStruct((B, N, C), jnp.float32),
            jax.ShapeDtypeStruct((B, 2, C), jnp.float32),
        ],
    )(G, X, W1, stats1, W2, gamma, beta, gmat)


def _edge_minmax_body(g_ref, c_ref, w1_ref, emax_ref, emin_ref, st_ref):
    t = pl.program_id(1)
    C = w1_ref.shape[1]
    w1b = w1_ref[...].astype(jnp.bfloat16)
    s = jnp.zeros((1, C), jnp.float32)
    sq = jnp.zeros((1, C), jnp.float32)
    emax = None
    emin = None
    for j in range(K):
        e = _edge_conv1_j(g_ref, c_ref, w1b, j)
        s = s + jnp.sum(e, axis=0, keepdims=True)
        sq = sq + jnp.sum(e * e, axis=0, keepdims=True)
        emax = e if emax is None else jnp.maximum(emax, e)
        emin = e if emin is None else jnp.minimum(emin, e)
    emax_ref[0] = emax
    emin_ref[0] = emin

    @pl.when(t == 0)
    def _():
        st_ref[0] = jnp.zeros_like(st_ref[0])

    st_ref[0, 0:1, :] += s
    st_ref[0, 1:2, :] += sq


def _edge_minmax(G, X, W1):
    B = G.shape[0]
    Gc = G.shape[3]
    Ci = X.shape[2]
    C = W1.shape[1]
    return pl.pallas_call(
        _edge_minmax_body,
        grid=(B, NT),
        in_specs=[
            pl.BlockSpec((1, TN, K, Gc), lambda b, t: (b, t, 0, 0)),
            pl.BlockSpec((1, TN, Ci), lambda b, t: (b, t, 0)),
            pl.BlockSpec((2 * Ci, C), lambda b, t: (0, 0)),
        ],
        out_specs=[
            pl.BlockSpec((1, TN, C), lambda b, t: (b, t, 0)),
            pl.BlockSpec((1, TN, C), lambda b, t: (b, t, 0)),
            pl.BlockSpec((1, 2, C), lambda b, t: (b, 0, 0)),
        ],
        out_shape=[
            jax.ShapeDtypeStruct((B, N, C), jnp.float32),
            jax.ShapeDtypeStruct((B, N, C), jnp.float32),
            jax.ShapeDtypeStruct((B, 2, C), jnp.float32),
        ],
    )(G, X, W1)


def _finalize_body(cmax_ref, cmin_ref, st_ref, gam_ref, bet_ref,
                   gmat_ref, x_ref):
    C = cmax_ref.shape[2]
    sc, sh = _gn_affine(st_ref[0, 0:1, :], st_ref[0, 1:2, :],
                        gam_ref[...], bet_ref[...], gmat_ref[...],
                        (C // 32) * N * K)
    sel = jnp.where(sc >= 0, cmax_ref[0], cmin_ref[0])
    x_ref[0] = _lrelu(sel * sc + sh)


def _finalize(cmax, cmin, stats, gamma, beta, gmat):
    B = cmax.shape[0]
    C = cmax.shape[2]
    return pl.pallas_call(
        _finalize_body,
        grid=(B, NT),
        in_specs=[
            pl.BlockSpec((1, TN, C), lambda b, t: (b, t, 0)),
            pl.BlockSpec((1, TN, C), lambda b, t: (b, t, 0)),
            pl.BlockSpec((1, 2, C), lambda b, t: (b, 0, 0)),
            pl.BlockSpec((1, C), lambda b, t: (0, 0)),
            pl.BlockSpec((1, C), lambda b, t: (0, 0)),
            pl.BlockSpec((C, C), lambda b, t: (0, 0)),
        ],
        out_specs=pl.BlockSpec((1, TN, C), lambda b, t: (b, t, 0)),
        out_shape=jax.ShapeDtypeStruct((B, N, C), jnp.float32),
    )(cmax, cmin, stats, gamma, beta, gmat)


def _f1_body(x1_ref, x2_ref, emax_ref, emin_ref, st5_ref, g5_ref, b5_ref,
             gm64_ref, w6_ref, w7b_ref,
             c7p_ref, st6_ref, ext6_ref, st7_ref):
    t = pl.program_id(1)
    sc5, sh5 = _gn_affine(st5_ref[0, 0:1, :], st5_ref[0, 1:2, :],
                          g5_ref[...], b5_ref[...], gm64_ref[...],
                          2 * N * K)
    sel = jnp.where(sc5 >= 0, emax_ref[0], emin_ref[0])
    x3 = _lrelu(sel * sc5 + sh5)
    xcat = jnp.concatenate([x1_ref[0], x2_ref[0], x3], axis=1)
    c6 = jnp.dot(xcat, w6_ref[...], preferred_element_type=jnp.float32, precision=lax.Precision.HIGHEST)
    c7p = jnp.dot(xcat, w7b_ref[...], preferred_element_type=jnp.float32, precision=lax.Precision.HIGHEST)
    c7p_ref[0] = c7p
    s6 = jnp.sum(c6, axis=0, keepdims=True)
    q6 = jnp.sum(c6 * c6, axis=0, keepdims=True)
    m6 = jnp.max(c6, axis=0, keepdims=True)
    n6 = jnp.min(c6, axis=0, keepdims=True)
    s7 = jnp.sum(c7p, axis=0, keepdims=True)
    q7 = jnp.sum(c7p * c7p, axis=0, keepdims=True)

    @pl.when(t == 0)
    def _():
        st6_ref[0] = jnp.zeros_like(st6_ref[0])
        st7_ref[0] = jnp.zeros_like(st7_ref[0])
        ext6_ref[0, 0:1, :] = jnp.full_like(m6, -jnp.inf)
        ext6_ref[0, 1:2, :] = jnp.full_like(n6, jnp.inf)

    st6_ref[0, 0:1, :] += s6
    st6_ref[0, 1:2, :] += q6
    st7_ref[0, 0:1, :] += s7
    st7_ref[0, 1:2, :] += q7
    ext6_ref[0, 0:1, :] = jnp.maximum(ext6_ref[0, 0:1, :], m6)
    ext6_ref[0, 1:2, :] = jnp.minimum(ext6_ref[0, 1:2, :], n6)


def _f1(x1, x2, emax5, emin5, st5, g5, b5, gm64, W6, W7b):
    B = x1.shape[0]
    emb = W6.shape[1]
    c7 = W7b.shape[1]
    return pl.pallas_call(
        _f1_body,
        grid=(B, NT),
        in_specs=[
            pl.BlockSpec((1, TN, 64), lambda b, t: (b, t, 0)),
            pl.BlockSpec((1, TN, 64), lambda b, t: (b, t, 0)),
            pl.BlockSpec((1, TN, 64), lambda b, t: (b, t, 0)),
            pl.BlockSpec((1, TN, 64), lambda b, t: (b, t, 0)),
            pl.BlockSpec((1, 2, 64), lambda b, t: (b, 0, 0)),
            pl.BlockSpec((1, 64), lambda b, t: (0, 0)),
            pl.BlockSpec((1, 64), lambda b, t: (0, 0)),
            pl.BlockSpec((64, 64), lambda b, t: (0, 0)),
            pl.BlockSpec((192, emb), lambda b, t: (0, 0)),
            pl.BlockSpec((192, c7), lambda b, t: (0, 0)),
        ],
        out_specs=[
            pl.BlockSpec((1, TN, c7), lambda b, t: (b, t, 0)),
            pl.BlockSpec((1, 2, emb), lambda b, t: (b, 0, 0)),
            pl.BlockSpec((1, 2, emb), lambda b, t: (b, 0, 0)),
            pl.BlockSpec((1, 2, c7), lambda b, t: (b, 0, 0)),
        ],
        out_shape=[
            jax.ShapeDtypeStruct((B, N, c7), jnp.float32),
            jax.ShapeDtypeStruct((B, 2, emb), jnp.float32),
            jax.ShapeDtypeStruct((B, 2, emb), jnp.float32),
            jax.ShapeDtypeStruct((B, 2, c7), jnp.float32),
        ],
    )(x1, x2, emax5, emin5, st5, g5, b5, gm64, W6, W7b)


def _f2_body(ext6_ref, st6_ref, st7_ref, w7a_ref, g6_ref, b6_ref,
             g7_ref, b7_ref, gm6_ref, gm7_ref, aff_ref):
    emb = st6_ref.shape[2]
    c7 = st7_ref.shape[2]
    sc6, sh6 = _gn_affine(st6_ref[0, 0:1, :], st6_ref[0, 1:2, :],
                          g6_ref[...], b6_ref[...], gm6_ref[...],
                          (emb // 32) * N)
    hglob = _lrelu(jnp.where(sc6 >= 0, ext6_ref[0, 0:1, :],
                             ext6_ref[0, 1:2, :]) * sc6 + sh6)
    k7 = jnp.dot(hglob, w7a_ref[...], preferred_element_type=jnp.float32, precision=lax.Precision.HIGHEST)
    s7 = st7_ref[0, 0:1, :] + N * k7
    q7 = st7_ref[0, 1:2, :] + 2.0 * k7 * st7_ref[0, 0:1, :] + N * k7 * k7
    sc7, sh7 = _gn_affine(s7, q7, g7_ref[...], b7_ref[...], gm7_ref[...],
                          (c7 // 32) * N)
    aff_ref[0, 0:1, :] = sc7
    aff_ref[0, 1:2, :] = sh7 + k7 * sc7


def _f2(ext6, st6, st7, W7a, g6, b6, g7, b7, gm6, gm7):
    B = ext6.shape[0]
    emb = st6.shape[2]
    c7 = st7.shape[2]
    return pl.pallas_call(
        _f2_body,
        grid=(B,),
        in_specs=[
            pl.BlockSpec((1, 2, emb), lambda b: (b, 0, 0)),
            pl.BlockSpec((1, 2, emb), lambda b: (b, 0, 0)),
            pl.BlockSpec((1, 2, c7), lambda b: (b, 0, 0)),
            pl.BlockSpec((emb, c7), lambda b: (0, 0)),
            pl.BlockSpec((1, emb), lambda b: (0, 0)),
            pl.BlockSpec((1, emb), lambda b: (0, 0)),
            pl.BlockSpec((1, c7), lambda b: (0, 0)),
            pl.BlockSpec((1, c7), lambda b: (0, 0)),
            pl.BlockSpec((emb, emb), lambda b: (0, 0)),
            pl.BlockSpec((c7, c7), lambda b: (0, 0)),
        ],
        out_specs=pl.BlockSpec((1, 2, c7), lambda b: (b, 0, 0)),
        out_shape=jax.ShapeDtypeStruct((B, 2, c7), jnp.float32),
    )(ext6, st6, st7, W7a, g6, b6, g7, b7, gm6, gm7)


def _f3_body(c7p_ref, aff_ref, w8_ref, c8_ref, st8_ref):
    t = pl.program_id(1)
    h7 = _lrelu(c7p_ref[0] * aff_ref[0, 0:1, :] + aff_ref[0, 1:2, :])
    c8 = jnp.dot(h7, w8_ref[...], preferred_element_type=jnp.float32, precision=lax.Precision.HIGHEST)
    c8_ref[0] = c8

    @pl.when(t == 0)
    def _():
        st8_ref[0] = jnp.zeros_like(st8_ref[0])

    st8_ref[0, 0:1, :] += jnp.sum(c8, axis=0, keepdims=True)
    st8_ref[0, 1:2, :] += jnp.sum(c8 * c8, axis=0, keepdims=True)


def _f3(c7p, aff7, W8):
    B = c7p.shape[0]
    c7 = c7p.shape[2]
    c8 = W8.shape[1]
    return pl.pallas_call(
        _f3_body,
        grid=(B, NT),
        in_specs=[
            pl.BlockSpec((1, TN, c7), lambda b, t: (b, t, 0)),
            pl.BlockSpec((1, 2, c7), lambda b, t: (b, 0, 0)),
            pl.BlockSpec((c7, c8), lambda b, t: (0, 0)),
        ],
        out_specs=[
            pl.BlockSpec((1, TN, c8), lambda b, t: (b, t, 0)),
            pl.BlockSpec((1, 2, c8), lambda b, t: (b, 0, 0)),
        ],
        out_shape=[
            jax.ShapeDtypeStruct((B, N, c8), jnp.float32),
            jax.ShapeDtypeStruct((B, 2, c8), jnp.float32),
        ],
    )(c7p, aff7, W8)


def _f4_body(c8_ref, st8_ref, g8_ref, b8_ref, gm8_ref, w9_ref, w10_ref,
             seg_ref, key_ref):
    C = c8_ref.shape[2]
    sc8, sh8 = _gn_affine(st8_ref[0, 0:1, :], st8_ref[0, 1:2, :],
                          g8_ref[...], b8_ref[...], gm8_ref[...],
                          (C // 32) * N)
    h8 = _lrelu(c8_ref[0] * sc8 + sh8)
    seg_ref[0] = jnp.dot(h8, w9_ref[...], preferred_element_type=jnp.float32, precision=lax.Precision.HIGHEST)
    key_ref[0] = jnp.dot(h8, w10_ref[...], preferred_element_type=jnp.float32, precision=lax.Precision.HIGHEST)


def _f4(c8, st8, g8, b8, gm8, W9, W10):
    B = c8.shape[0]
    C = c8.shape[2]
    o1 = W9.shape[1]
    o2 = W10.shape[1]
    return pl.pallas_call(
        _f4_body,
        grid=(B, NT),
        in_specs=[
            pl.BlockSpec((1, TN, C), lambda b, t: (b, t, 0)),
            pl.BlockSpec((1, 2, C), lambda b, t: (b, 0, 0)),
            pl.BlockSpec((1, C), lambda b, t: (0, 0)),
            pl.BlockSpec((1, C), lambda b, t: (0, 0)),
            pl.BlockSpec((C, C), lambda b, t: (0, 0)),
            pl.BlockSpec((C, o1), lambda b, t: (0, 0)),
            pl.BlockSpec((C, o2), lambda b, t: (0, 0)),
        ],
        out_specs=[
            pl.BlockSpec((1, TN, o1), lambda b, t: (b, t, 0)),
            pl.BlockSpec((1, TN, o2), lambda b, t: (b, t, 0)),
        ],
        out_shape=[
            jax.ShapeDtypeStruct((B, N, o1), jnp.float32),
            jax.ShapeDtypeStruct((B, N, o2), jnp.float32),
        ],
    )(c8, st8, g8, b8, gm8, W9, W10)


def _group_mat(C):
    g = jnp.repeat(jnp.arange(32), C // 32)
    return (g[:, None] == g[None, :]).astype(jnp.float32)


def kernel(x, device, W1, g1, b1, W2, g2, b2, W3, g3, b3, W4, g4, b4,
           W5, g5, b5, W6, g6, b6, W7, g7, b7, W8, g8, b8, W9, W10):
    B = x.shape[0]
    xt = jnp.transpose(x, (0, 2, 1))
    xt3 = xt[:, :, :3]
    gm64 = _group_mat(64)
    emb = W6.shape[1]
    gm_emb = _group_mat(emb)
    gm512 = _group_mat(W7.shape[1])
    gm256 = _group_mat(W8.shape[1])
    r2 = lambda v: v.reshape(1, -1)

    def pad128(a):
        Bb, n, c = a.shape
        z = jnp.zeros((Bb, n, 128 - c), jnp.float32)
        return jnp.concatenate([a, z], axis=2).reshape(Bb * n, 128)

    def edge_block2(xt_in, idx, W1_, W2_, ga, ba, gb, bb):
        G = _sc_gather(pad128(xt_in), idx.reshape(-1))
        G = G.reshape(B, N, K, 128)
        st1 = _edge_stats(G, xt_in, W1_)
        cmax, cmin, st2 = _edge_conv2(G, xt_in, W1_, st1, W2_,
                                      r2(ga), r2(ba), gm64)
        return _finalize(cmax, cmin, st2, r2(gb), r2(bb), gm64)

    idx1 = _knn(xt3)
    x1 = edge_block2(xt, idx1, W1, W2, g1, b1, g2, b2)

    idx2 = _knn(x1)
    x2 = edge_block2(x1, idx2, W3, W4, g3, b3, g4, b4)

    idx3 = _knn(x2)
    G3 = _sc_gather(pad128(x2), idx3.reshape(-1))
    G3 = G3.reshape(B, N, K, 128)
    emax5, emin5, st5 = _edge_minmax(G3, x2, W5)

    c7p, st6, ext6, st7 = _f1(x1, x2, emax5, emin5, st5, r2(g5), r2(b5),
                              gm64, W6, W7[emb:])
    aff7 = _f2(ext6, st6, st7, W7[:emb], r2(g6), r2(b6), r2(g7), r2(b7),
               gm_emb, gm512)
    c8, st8 = _f3(c7p, aff7, W8)
    seg, key = _f4(c8, st8, r2(g8), r2(b8), gm256, W9, W10)
    return (jnp.transpose(seg, (0, 2, 1)), jnp.transpose(key, (0, 2, 1)))
